# Initial kernel scaffold; baseline (speedup 1.0000x reference)
import jax
import jax.numpy as jnp
from jax.experimental import pallas as pl


def kernel(x, W, b):
    raise NotImplementedError("write your pallas kernel here")


# speedup vs baseline: 3.2374x; 3.2374x over previous
import functools

import jax
import jax.numpy as jnp
from jax.experimental import pallas as pl


_TILE = 1024


def _router_body(x_ref, wt_ref, b_ref, disp_ref, cnt_ref):
    logits = jnp.dot(
        x_ref[...], wt_ref[...],
        preferred_element_type=jnp.float32,
    ) + b_ref[...]
    m = jnp.max(logits, axis=-1, keepdims=True)
    e = jnp.exp(logits - m)
    scores = e / jnp.sum(e, axis=-1, keepdims=True)
    idx = jax.lax.broadcasted_iota(jnp.int32, scores.shape, 1)
    m1 = jnp.max(scores, axis=-1, keepdims=True)
    i1 = jnp.min(jnp.where(scores == m1, idx, 16), axis=-1, keepdims=True)
    mask1 = idx == i1
    rest = jnp.where(mask1, -1.0, scores)
    m2 = jnp.max(rest, axis=-1, keepdims=True)
    i2 = jnp.min(jnp.where(rest == m2, idx, 16), axis=-1, keepdims=True)
    disp = jnp.where(mask1 | (idx == i2), scores, 0.0)
    disp_ref[...] = disp

    @pl.when(pl.program_id(0) == 0)
    def _init():
        cnt_ref[...] = jnp.zeros_like(cnt_ref)

    cnt_ref[...] += jnp.sum(disp, axis=0, keepdims=True)


@functools.partial(jax.jit, static_argnames=())
def kernel(x, W, b):
    B, S, D = x.shape
    E = W.shape[0]
    n_tokens = B * S
    xf = x.reshape(n_tokens, D)
    wt = W.T
    b2 = b.reshape(1, E)
    grid = (n_tokens // _TILE,)
    disp, cnt = pl.pallas_call(
        _router_body,
        grid=grid,
        in_specs=[
            pl.BlockSpec((_TILE, D), lambda i: (i, 0)),
            pl.BlockSpec((D, E), lambda i: (0, 0)),
            pl.BlockSpec((1, E), lambda i: (0, 0)),
        ],
        out_specs=[
            pl.BlockSpec((_TILE, E), lambda i: (i, 0)),
            pl.BlockSpec((1, E), lambda i: (0, 0)),
        ],
        out_shape=[
            jax.ShapeDtypeStruct((n_tokens, E), jnp.float32),
            jax.ShapeDtypeStruct((1, E), jnp.float32),
        ],
    )(xf, wt, b2)
    dispatch = disp.reshape(B, S, E)
    return (dispatch, dispatch, cnt.reshape(E))


# speedup vs baseline: 3.3899x; 1.0471x over previous
import functools

import jax
import jax.numpy as jnp
from jax.experimental import pallas as pl


_TILE = 2048


def _router_body(x_ref, wt_ref, b_ref, disp_ref, cnt_ref):
    logits = jnp.dot(
        x_ref[...], wt_ref[...],
        preferred_element_type=jnp.float32,
    ) + b_ref[...]
    m = jnp.max(logits, axis=-1, keepdims=True)
    e = jnp.exp(logits - m)
    scores = e / jnp.sum(e, axis=-1, keepdims=True)
    idx = jax.lax.broadcasted_iota(jnp.int32, scores.shape, 1)
    m1 = jnp.max(scores, axis=-1, keepdims=True)
    i1 = jnp.min(jnp.where(scores == m1, idx, 16), axis=-1, keepdims=True)
    mask1 = idx == i1
    rest = jnp.where(mask1, -1.0, scores)
    m2 = jnp.max(rest, axis=-1, keepdims=True)
    i2 = jnp.min(jnp.where(rest == m2, idx, 16), axis=-1, keepdims=True)
    disp = jnp.where(mask1 | (idx == i2), scores, 0.0)
    disp_ref[...] = disp

    @pl.when(pl.program_id(0) == 0)
    def _init():
        cnt_ref[...] = jnp.zeros_like(cnt_ref)

    cnt_ref[...] += jnp.sum(disp, axis=0, keepdims=True)


@functools.partial(jax.jit, static_argnames=())
def kernel(x, W, b):
    B, S, D = x.shape
    E = W.shape[0]
    n_tokens = B * S
    xf = x.reshape(n_tokens, D)
    wt = W.T
    b2 = b.reshape(1, E)
    grid = (n_tokens // _TILE,)
    disp, cnt = pl.pallas_call(
        _router_body,
        grid=grid,
        in_specs=[
            pl.BlockSpec((_TILE, D), lambda i: (i, 0)),
            pl.BlockSpec((D, E), lambda i: (0, 0)),
            pl.BlockSpec((1, E), lambda i: (0, 0)),
        ],
        out_specs=[
            pl.BlockSpec((_TILE, E), lambda i: (i, 0)),
            pl.BlockSpec((1, E), lambda i: (0, 0)),
        ],
        out_shape=[
            jax.ShapeDtypeStruct((n_tokens, E), jnp.float32),
            jax.ShapeDtypeStruct((1, E), jnp.float32),
        ],
    )(xf, wt, b2)
    dispatch = disp.reshape(B, S, E)
    return (dispatch, dispatch, cnt.reshape(E))


# speedup vs baseline: 3.4247x; 1.0103x over previous
import functools

import jax
import jax.numpy as jnp
from jax.experimental import pallas as pl


_TILE = 2048


def _router_body(x_ref, wt_ref, b_ref, disp_ref, cnt_ref):
    logits = jnp.dot(
        x_ref[...], wt_ref[...],
        preferred_element_type=jnp.float32,
    )
    lt = logits.T + b_ref[...]
    m = jnp.max(lt, axis=0, keepdims=True)
    e = jnp.exp(lt - m)
    scores = e / jnp.sum(e, axis=0, keepdims=True)
    idx = jax.lax.broadcasted_iota(jnp.int32, scores.shape, 0)
    m1 = jnp.max(scores, axis=0, keepdims=True)
    i1 = jnp.min(jnp.where(scores == m1, idx, 16), axis=0, keepdims=True)
    mask1 = idx == i1
    rest = jnp.where(mask1, -1.0, scores)
    m2 = jnp.max(rest, axis=0, keepdims=True)
    i2 = jnp.min(jnp.where(rest == m2, idx, 16), axis=0, keepdims=True)
    disp_t = jnp.where(mask1 | (idx == i2), scores, 0.0)
    disp_ref[...] = disp_t.T

    @pl.when(pl.program_id(0) == 0)
    def _init():
        cnt_ref[...] = jnp.zeros_like(cnt_ref)

    cnt_ref[...] += jnp.sum(disp_t, axis=1, keepdims=True)


@functools.partial(jax.jit, static_argnames=())
def kernel(x, W, b):
    B, S, D = x.shape
    E = W.shape[0]
    n_tokens = B * S
    xf = x.reshape(n_tokens, D)
    wt = W.T
    bc = b.reshape(E, 1)
    grid = (n_tokens // _TILE,)
    disp, cnt = pl.pallas_call(
        _router_body,
        grid=grid,
        in_specs=[
            pl.BlockSpec((_TILE, D), lambda i: (i, 0)),
            pl.BlockSpec((D, E), lambda i: (0, 0)),
            pl.BlockSpec((E, 1), lambda i: (0, 0)),
        ],
        out_specs=[
            pl.BlockSpec((_TILE, E), lambda i: (i, 0)),
            pl.BlockSpec((E, 1), lambda i: (0, 0)),
        ],
        out_shape=[
            jax.ShapeDtypeStruct((n_tokens, E), jnp.float32),
            jax.ShapeDtypeStruct((E, 1), jnp.float32),
        ],
    )(xf, wt, bc)
    dispatch = disp.reshape(B, S, E)
    return (dispatch, dispatch, cnt.reshape(E))
